# baseline (speedup 1.0000x reference)
import jax
import jax.numpy as jnp
from jax import lax
from jax.experimental import pallas as pl

B, S, A, G, NB, D = 16, 1024, 64, 128, 2048, 256
F3 = 3 * G


def _stage1_body(subsT_ref, vf_ref, eoh_ref, mask_ref,
                 W_ih_ref, W_hh_ref, b_ih_ref, b_hh_ref,
                 ln_sub_g_ref, ln_sub_b_ref, ln_post_g_ref, ln_post_b_ref,
                 l1_W_ref, l1_b_ref, l2_W_ref, l2_b_ref,
                 ln_pre_g_ref, ln_pre_b_ref, score_W_ref, score_b_ref,
                 probs_ref):
    f32 = jnp.float32
    cT = (((0,), (0,)), ((), ()))
    subsT = subsT_ref[0]
    subsT_m = subsT * mask_ref[0]
    vf = vf_ref[0]

    swsT = lax.dot_general(vf, subsT_m, cT, preferred_element_type=f32)
    inv_size = 1.0 / (jnp.sum(subsT_m, axis=0, keepdims=True) + 1e-4)
    meanT = swsT * inv_size

    m = jnp.mean(meanT, axis=0, keepdims=True)
    v = jnp.mean((meanT - m) ** 2, axis=0, keepdims=True)
    hT = ((meanT - m) * lax.rsqrt(v + 1e-5) * ln_sub_g_ref[...]
          + ln_sub_b_ref[...])

    r8 = lax.broadcasted_iota(jnp.int32, (8, G), 0)
    c8 = lax.broadcasted_iota(jnp.int32, (8, G), 1)
    P8 = jnp.where((c8 // 20 == r8) & (c8 < 100), 1.0, 0.0).astype(f32)
    EP = jnp.dot(eoh_ref[0], P8, preferred_element_type=f32)
    TT = lax.dot_general(EP, subsT, cT, preferred_element_type=f32)
    row1 = lax.broadcasted_iota(jnp.int32, (G, 1), 0)
    rowmod = (row1 % 20).astype(f32)
    valid = row1 < 100
    xT = jnp.where((rowmod <= TT) & valid, 1.0, 0.0)

    gi = jnp.dot(W_ih_ref[...], xT, preferred_element_type=f32) + b_ih_ref[...]
    gh = jnp.dot(W_hh_ref[...], hT, preferred_element_type=f32) + b_hh_ref[...]
    r = jax.nn.sigmoid(gi[:G] + gh[:G])
    z = jax.nn.sigmoid(gi[G:2 * G] + gh[G:2 * G])
    n = jnp.tanh(gi[2 * G:] + r * gh[2 * G:])
    combT = (1.0 - z) * n + z * hT

    m2 = jnp.mean(combT, axis=0, keepdims=True)
    v2 = jnp.mean((combT - m2) ** 2, axis=0, keepdims=True)
    yT = ((combT - m2) * lax.rsqrt(v2 + 1e-5) * ln_post_g_ref[...]
          + ln_post_b_ref[...])

    yT = jax.nn.relu(jnp.dot(l1_W_ref[...], yT, preferred_element_type=f32)
                     + l1_b_ref[...])
    yT = jax.nn.relu(jnp.dot(l2_W_ref[...], yT, preferred_element_type=f32)
                     + l2_b_ref[...])
    m3 = jnp.mean(yT, axis=0, keepdims=True)
    v3 = jnp.mean((yT - m3) ** 2, axis=0, keepdims=True)
    yT = ((yT - m3) * lax.rsqrt(v3 + 1e-5) * ln_pre_g_ref[...]
          + ln_pre_b_ref[...])

    scores = (jnp.dot(score_W_ref[...], yT, preferred_element_type=f32)
              + score_b_ref[...])

    e = jnp.exp(scores - jnp.max(scores, axis=1, keepdims=True))
    probs_ref[pl.ds(pl.program_id(0), 1), :] = e / jnp.sum(e, axis=1,
                                                           keepdims=True)


def _stage2_body(mm_ref, probs_ref, out_ref):
    b = pl.program_id(0)
    out_ref[pl.ds(b, 1), :] = lax.dot_general(
        probs_ref[pl.ds(b, 1), :], mm_ref[0], (((1,), (1,)), ((), ())),
        preferred_element_type=jnp.float32)


def kernel(vert_feat_in, vert_mask_in, vert_element_oh, adj_oh, atom_subsets,
           atom_subsets_peaks, sparse_mass_matrix, W_ih, W_hh, b_ih, b_hh,
           ln_sub_g, ln_sub_b, ln_post_g, ln_post_b, l1_W, l1_b, l2_W, l2_b,
           ln_pre_g, ln_pre_b, score_W, score_b):
    f32 = jnp.float32
    subsT = jnp.swapaxes(atom_subsets, 1, 2)
    maskc = vert_mask_in.reshape(B, A, 1)
    eoh8 = jnp.pad(vert_element_oh, ((0, 0), (0, 0), (0, 3)))
    W_ihp = jnp.pad(W_ih, ((0, 0), (0, G - 100)))
    col = lambda a: a.reshape(-1, 1)

    full = lambda shp: pl.BlockSpec(shp, lambda b: (0,) * len(shp))
    probs = pl.pallas_call(
        _stage1_body,
        grid=(B,),
        in_specs=[
            pl.BlockSpec((1, A, S), lambda b: (b, 0, 0)),
            pl.BlockSpec((1, A, G), lambda b: (b, 0, 0)),
            pl.BlockSpec((1, A, 8), lambda b: (b, 0, 0)),
            pl.BlockSpec((1, A, 1), lambda b: (b, 0, 0)),
            full((F3, G)), full((F3, G)), full((F3, 1)), full((F3, 1)),
            full((G, 1)), full((G, 1)), full((G, 1)), full((G, 1)),
            full((D, G)), full((D, 1)), full((D, D)), full((D, 1)),
            full((D, 1)), full((D, 1)), full((1, D)), full((1, 1)),
        ],
        out_specs=pl.BlockSpec((B, S), lambda b: (0, 0)),
        out_shape=jax.ShapeDtypeStruct((B, S), f32),
    )(subsT, vert_feat_in, eoh8, maskc,
      W_ihp, W_hh, col(b_ih), col(b_hh),
      col(ln_sub_g), col(ln_sub_b), col(ln_post_g), col(ln_post_b),
      l1_W, col(l1_b), l2_W, col(l2_b),
      col(ln_pre_g), col(ln_pre_b), score_W, score_b.reshape(1, 1))

    BN = 2048
    spect = pl.pallas_call(
        _stage2_body,
        grid=(B, NB // BN),
        in_specs=[
            pl.BlockSpec((1, BN, S), lambda b, n: (b, n, 0)),
            pl.BlockSpec((B, S), lambda b, n: (0, 0)),
        ],
        out_specs=pl.BlockSpec((B, NB), lambda b, n: (0, 0)),
        out_shape=jax.ShapeDtypeStruct((B, NB), f32),
    )(sparse_mass_matrix, probs)

    return (spect, probs)
